# baseline (speedup 1.0000x reference)
import functools
import math

import jax
import jax.numpy as jnp
from jax import lax
from jax.experimental import pallas as pl
from jax.experimental.pallas import tpu as pltpu
from jax.experimental.pallas import tpu_sc as plsc

_NC = 2
_NS = 16
_NW = _NC * _NS

_ROWS_PER_BLOCK = 6144
_NBUF = 2


def _proj_body(scale_ref, tt_ref, w_ref, b_ref, o_ref):
    acc = lax.dot_general(
        tt_ref[...],
        w_ref[...],
        (((0,), (0,)), ((), ())),
        preferred_element_type=jnp.float32,
    )
    o_ref[...] = (acc + b_ref[...]) * scale_ref[0]


@functools.partial(jax.jit, static_argnames=("d_model",))
def _project_table(table_t, W, b, scale, d_model):
    emb, vocab = table_t.shape
    blk = _ROWS_PER_BLOCK
    grid = (vocab + blk - 1) // blk
    return pl.pallas_call(
        _proj_body,
        grid=(grid,),
        in_specs=[
            pl.BlockSpec(memory_space=pltpu.SMEM),
            pl.BlockSpec((emb, blk), lambda i: (0, i)),
            pl.BlockSpec((emb, d_model), lambda i: (0, 0)),
            pl.BlockSpec((1, d_model), lambda i: (0, 0)),
        ],
        out_specs=pl.BlockSpec((blk, d_model), lambda i: (i, 0)),
        out_shape=jax.ShapeDtypeStruct((vocab, d_model), jnp.float32),
    )(scale, table_t, W, b.reshape(1, d_model))


@functools.lru_cache(maxsize=None)
def _make_gather(B, D, C):
    b_per_w = B // _NW
    n_chunks = b_per_w // C
    n_groups = n_chunks // _NBUF
    mesh = plsc.VectorSubcoreMesh(core_axis_name="c", subcore_axis_name="s")

    @functools.partial(
        pl.kernel,
        out_type=jax.ShapeDtypeStruct((B, D), jnp.float32),
        mesh=mesh,
        scratch_types=[
            pltpu.VMEM((n_chunks, C), jnp.int32),
            [pltpu.VMEM((C, D), jnp.float32)] * _NBUF,
            [pltpu.SemaphoreType.DMA] * _NBUF,
            [pltpu.SemaphoreType.DMA] * _NBUF,
        ],
    )
    def gather_kernel(p_hbm, idx_hbm, out_hbm, idx_v, bufs, gsems, ssems):
        wid = lax.axis_index("s") * _NC + lax.axis_index("c")
        base = wid * b_per_w
        pltpu.sync_copy(idx_hbm.at[wid], idx_v)

        for p in range(_NBUF):
            pltpu.async_copy(p_hbm.at[idx_v.at[p]], bufs[p], gsems[p])

        def body(g, carry):
            for p in range(_NBUF):
                j = g * _NBUF + p
                pltpu.make_async_copy(
                    p_hbm.at[idx_v.at[j]], bufs[p], gsems[p]
                ).wait()
                pltpu.async_copy(
                    bufs[p], out_hbm.at[pl.ds(base + j * C, C)], ssems[p]
                )
            for p in range(_NBUF):
                j = g * _NBUF + p
                jn = j + _NBUF
                pltpu.make_async_copy(
                    bufs[p], out_hbm.at[pl.ds(base + j * C, C)], ssems[p]
                ).wait()

                @pl.when(jn < n_chunks)
                def _():
                    pltpu.async_copy(p_hbm.at[idx_v.at[jn]], bufs[p], gsems[p])

            return carry

        lax.fori_loop(0, n_groups, body, 0)

    return gather_kernel


def kernel(x, table, W, b):
    batch, seq = x.shape
    d_model = W.shape[1]
    scale = jnp.full((1,), math.sqrt(d_model), dtype=jnp.float32)
    proj = _project_table(table.T, W, b, scale, d_model)

    B = batch * seq
    C = 80
    idx = x.reshape(_NW, (B // _NW) // C, C).astype(jnp.int32)
    out = _make_gather(B, d_model, C)(proj, idx)
    return out.reshape(batch, seq, d_model)


# speedup vs baseline: 1.0001x; 1.0001x over previous
import functools
import math

import jax
import jax.numpy as jnp
from jax import lax
from jax.experimental import pallas as pl
from jax.experimental.pallas import tpu as pltpu
from jax.experimental.pallas import tpu_sc as plsc

_NC = 2
_NS = 16
_NW = _NC * _NS

_ROWS_PER_BLOCK = 6144
_NBUF = 4


def _proj_body(scale_ref, tt_ref, w_ref, b_ref, o_ref):
    acc = lax.dot_general(
        tt_ref[...],
        w_ref[...],
        (((0,), (0,)), ((), ())),
        preferred_element_type=jnp.float32,
    )
    o_ref[...] = (acc + b_ref[...]) * scale_ref[0]


@functools.partial(jax.jit, static_argnames=("d_model",))
def _project_table(table_t, W, b, scale, d_model):
    emb, vocab = table_t.shape
    blk = _ROWS_PER_BLOCK
    grid = (vocab + blk - 1) // blk
    return pl.pallas_call(
        _proj_body,
        grid=(grid,),
        in_specs=[
            pl.BlockSpec(memory_space=pltpu.SMEM),
            pl.BlockSpec((emb, blk), lambda i: (0, i)),
            pl.BlockSpec((emb, d_model), lambda i: (0, 0)),
            pl.BlockSpec((1, d_model), lambda i: (0, 0)),
        ],
        out_specs=pl.BlockSpec((blk, d_model), lambda i: (i, 0)),
        out_shape=jax.ShapeDtypeStruct((vocab, d_model), jnp.float32),
    )(scale, table_t, W, b.reshape(1, d_model))


@functools.lru_cache(maxsize=None)
def _make_gather(B, D, C):
    b_per_w = B // _NW
    n_chunks = b_per_w // C
    n_groups = n_chunks // _NBUF
    mesh = plsc.VectorSubcoreMesh(core_axis_name="c", subcore_axis_name="s")

    @functools.partial(
        pl.kernel,
        out_type=jax.ShapeDtypeStruct((B, D), jnp.float32),
        mesh=mesh,
        scratch_types=[
            pltpu.VMEM((n_chunks, C), jnp.int32),
            [pltpu.VMEM((C, D), jnp.float32)] * _NBUF,
            [pltpu.SemaphoreType.DMA] * _NBUF,
            [pltpu.SemaphoreType.DMA] * _NBUF,
        ],
    )
    def gather_kernel(p_hbm, idx_hbm, out_hbm, idx_v, bufs, gsems, ssems):
        wid = lax.axis_index("s") * _NC + lax.axis_index("c")
        base = wid * b_per_w
        pltpu.sync_copy(idx_hbm.at[wid], idx_v)

        for p in range(_NBUF):
            pltpu.async_copy(p_hbm.at[idx_v.at[p]], bufs[p], gsems[p])

        def body(g, carry):
            for p in range(_NBUF):
                j = g * _NBUF + p
                pltpu.make_async_copy(
                    p_hbm.at[idx_v.at[j]], bufs[p], gsems[p]
                ).wait()
                pltpu.async_copy(
                    bufs[p], out_hbm.at[pl.ds(base + j * C, C)], ssems[p]
                )
            for p in range(_NBUF):
                j = g * _NBUF + p
                jn = j + _NBUF
                pltpu.make_async_copy(
                    bufs[p], out_hbm.at[pl.ds(base + j * C, C)], ssems[p]
                ).wait()

                @pl.when(jn < n_chunks)
                def _():
                    pltpu.async_copy(p_hbm.at[idx_v.at[jn]], bufs[p], gsems[p])

            return carry

        lax.fori_loop(0, n_groups, body, 0)

    return gather_kernel


def kernel(x, table, W, b):
    batch, seq = x.shape
    d_model = W.shape[1]
    scale = jnp.full((1,), math.sqrt(d_model), dtype=jnp.float32)
    proj = _project_table(table.T, W, b, scale, d_model)

    B = batch * seq
    C = 32
    idx = x.reshape(_NW, (B // _NW) // C, C).astype(jnp.int32)
    out = _make_gather(B, d_model, C)(proj, idx)
    return out.reshape(batch, seq, d_model)


# speedup vs baseline: 1.0036x; 1.0035x over previous
import functools
import math

import jax
import jax.numpy as jnp
from jax import lax
from jax.experimental import pallas as pl
from jax.experimental.pallas import tpu as pltpu
from jax.experimental.pallas import tpu_sc as plsc

_NC = 2
_NS = 16
_NW = _NC * _NS

_ROWS_PER_BLOCK = 7168
_NBUF = 4


def _proj_body(scale_ref, tt_ref, w_ref, b_ref, o_ref):
    acc = lax.dot_general(
        tt_ref[...],
        w_ref[...],
        (((0,), (0,)), ((), ())),
        preferred_element_type=jnp.float32,
    )
    o_ref[...] = (acc + b_ref[...]) * scale_ref[0]


@functools.partial(jax.jit, static_argnames=("d_model",))
def _project_table(table_t, W, b, scale, d_model):
    emb, vocab = table_t.shape
    blk = _ROWS_PER_BLOCK
    grid = (vocab + blk - 1) // blk
    return pl.pallas_call(
        _proj_body,
        grid=(grid,),
        in_specs=[
            pl.BlockSpec(memory_space=pltpu.SMEM),
            pl.BlockSpec((emb, blk), lambda i: (0, i)),
            pl.BlockSpec((emb, d_model), lambda i: (0, 0)),
            pl.BlockSpec((1, d_model), lambda i: (0, 0)),
        ],
        out_specs=pl.BlockSpec((blk, d_model), lambda i: (i, 0)),
        out_shape=jax.ShapeDtypeStruct((vocab, d_model), jnp.float32),
    )(scale, table_t, W, b.reshape(1, d_model))


@functools.lru_cache(maxsize=None)
def _make_gather(B, D, C):
    b_per_w = B // _NW
    n_chunks = b_per_w // C
    n_groups = n_chunks // _NBUF
    mesh = plsc.VectorSubcoreMesh(core_axis_name="c", subcore_axis_name="s")

    @functools.partial(
        pl.kernel,
        out_type=jax.ShapeDtypeStruct((B, D), jnp.float32),
        mesh=mesh,
        scratch_types=[
            pltpu.VMEM((n_chunks, C), jnp.int32),
            [pltpu.VMEM((C, D), jnp.float32)] * _NBUF,
            [pltpu.SemaphoreType.DMA] * _NBUF,
            [pltpu.SemaphoreType.DMA] * _NBUF,
        ],
    )
    def gather_kernel(p_hbm, idx_hbm, out_hbm, idx_v, bufs, gsems, ssems):
        wid = lax.axis_index("s") * _NC + lax.axis_index("c")
        base = wid * b_per_w
        pltpu.sync_copy(idx_hbm.at[wid], idx_v)

        for p in range(_NBUF):
            pltpu.async_copy(p_hbm.at[idx_v.at[p]], bufs[p], gsems[p])

        def body(g, carry):
            for p in range(_NBUF):
                j = g * _NBUF + p
                pltpu.make_async_copy(
                    p_hbm.at[idx_v.at[j]], bufs[p], gsems[p]
                ).wait()
                pltpu.async_copy(
                    bufs[p], out_hbm.at[pl.ds(base + j * C, C)], ssems[p]
                )
            for p in range(_NBUF):
                j = g * _NBUF + p
                jn = j + _NBUF
                pltpu.make_async_copy(
                    bufs[p], out_hbm.at[pl.ds(base + j * C, C)], ssems[p]
                ).wait()

                @pl.when(jn < n_chunks)
                def _():
                    pltpu.async_copy(p_hbm.at[idx_v.at[jn]], bufs[p], gsems[p])

            return carry

        lax.fori_loop(0, n_groups, body, 0)

    return gather_kernel


def kernel(x, table, W, b):
    batch, seq = x.shape
    d_model = W.shape[1]
    scale = jnp.full((1,), math.sqrt(d_model), dtype=jnp.float32)
    proj = _project_table(table.T, W, b, scale, d_model)

    B = batch * seq
    C = 32
    idx = x.reshape(_NW, (B // _NW) // C, C).astype(jnp.int32)
    out = _make_gather(B, d_model, C)(proj, idx)
    return out.reshape(batch, seq, d_model)
